# baseline (speedup 1.0000x reference)
import functools

import jax
import jax.numpy as jnp
from jax import lax
from jax.experimental import pallas as pl
from jax.experimental.pallas import tpu as pltpu
from jax.experimental.pallas import tpu_sc as plsc

_NUM_CORES = 2
_NUM_SUBCORES = 16
_NUM_WORKERS = _NUM_CORES * _NUM_SUBCORES
_CHUNK = 128


@functools.lru_cache(maxsize=None)
def _make_kernel(V, D, B):
    b_per_w = B // _NUM_WORKERS
    n_chunks = b_per_w // _CHUNK
    mesh = plsc.VectorSubcoreMesh(
        core_axis_name="c",
        subcore_axis_name="s",
        num_cores=_NUM_CORES,
        num_subcores=_NUM_SUBCORES,
    )

    @functools.partial(
        pl.kernel,
        mesh=mesh,
        compiler_params=pltpu.CompilerParams(use_tc_tiling_on_sc=False),
        out_type=jax.ShapeDtypeStruct((B, D), jnp.float32),
        scratch_types=[
            pltpu.VMEM((n_chunks, _CHUNK), jnp.int32),
            pltpu.VMEM((b_per_w, D), jnp.float32),
            pltpu.SemaphoreType.DMA,
        ],
    )
    def gather_kernel(idx_hbm, table_hbm, out_hbm, idx_v, rows_v, sem):
        wid = lax.axis_index("s") * _NUM_CORES + lax.axis_index("c")
        pltpu.sync_copy(idx_hbm.at[pl.ds(wid * n_chunks, n_chunks)], idx_v)
        copies = []
        for j in range(n_chunks):
            copies.append(
                pltpu.async_copy(
                    table_hbm.at[idx_v.at[j]],
                    rows_v.at[pl.ds(j * _CHUNK, _CHUNK)],
                    sem,
                )
            )
        for c in copies:
            c.wait()
        pltpu.sync_copy(rows_v, out_hbm.at[pl.ds(wid * b_per_w, b_per_w)])

    return gather_kernel


def kernel(provider_ids, table):
    (B,) = provider_ids.shape
    V, D = table.shape
    idx2d = provider_ids.astype(jnp.int32).reshape(B // _CHUNK, _CHUNK)
    return _make_kernel(V, D, B)(idx2d, table)


# speedup vs baseline: 2.3752x; 2.3752x over previous
import functools

import jax
import jax.numpy as jnp
from jax import lax
from jax.experimental import pallas as pl
from jax.experimental.pallas import tpu as pltpu
from jax.experimental.pallas import tpu_sc as plsc

_NUM_CORES = 2
_NUM_SUBCORES = 16
_NUM_WORKERS = _NUM_CORES * _NUM_SUBCORES
_GROUP = 16


@functools.lru_cache(maxsize=None)
def _make_kernel(T, D, B):
    b_per_w = B // _NUM_WORKERS
    n_groups = b_per_w // _GROUP
    row_bytes = D * 4
    mesh = plsc.VectorSubcoreMesh(
        core_axis_name="c",
        subcore_axis_name="s",
        num_cores=_NUM_CORES,
        num_subcores=_NUM_SUBCORES,
    )

    @functools.partial(
        pl.kernel,
        mesh=mesh,
        out_type=jax.ShapeDtypeStruct((B // 8, 8, D), jnp.float32),
        scratch_types=[
            pltpu.VMEM((b_per_w,), jnp.int32),
            pltpu.VMEM((b_per_w,), jnp.int32),
            pltpu.VMEM((b_per_w // 8, 8, D), jnp.float32),
            pltpu.SemaphoreType.DMA,
        ],
    )
    def gather_kernel(t_hbm, s_hbm, table_hbm, out_hbm, t_sm, s_sm, rows_v, sem):
        wid = lax.axis_index("s") * _NUM_CORES + lax.axis_index("c")
        base = wid * b_per_w
        pltpu.sync_copy(t_hbm.at[pl.ds(base, b_per_w)], t_sm)
        pltpu.sync_copy(s_hbm.at[pl.ds(base, b_per_w)], s_sm)

        def step(g, _):
            tv = t_sm[pl.ds(g * _GROUP, _GROUP)]
            sv = s_sm[pl.ds(g * _GROUP, _GROUP)]
            copies = []
            for u in range(_GROUP):
                b = g * _GROUP + u
                copies.append(
                    pltpu.async_copy(
                        table_hbm.at[tv[u], sv[u]],
                        rows_v.at[b // 8, b % 8],
                        sem,
                    )
                )
            for c in copies:
                c.wait()
            return 0

        lax.fori_loop(0, n_groups, step, 0)
        pltpu.sync_copy(rows_v, out_hbm.at[pl.ds(base // 8, b_per_w // 8)])

    return gather_kernel


def kernel(provider_ids, table):
    (B,) = provider_ids.shape
    V, D = table.shape
    idx = provider_ids.astype(jnp.int32)
    t = idx // 8
    s = idx - t * 8
    table3 = table.reshape(V // 8, 8, D)
    out3 = _make_kernel(V // 8, D, B)(t, s, table3)
    return out3.reshape(B, D)
